# Initial kernel scaffold; baseline (speedup 1.0000x reference)
import jax
import jax.numpy as jnp
from jax.experimental import pallas as pl


def kernel(target_user, item_sample, user_sample, ui_src, ui_dst, ui_w, pi_src, pi_dst, pi_w, up_src, up_dst, up_w, embed_W, embed_pi_W, embed_u_W, W_se, b_se, W_te1, b_te1, W_te2, b_te2, gate1_W, gate1_b, gate2_W, gate2_b, t1_W1, t1_b1, t1_W2, t1_b2, t2_W1, t2_b1, t2_W2, t2_b2):
    raise NotImplementedError("write your pallas kernel here")


# speedup vs baseline: 2.4058x; 2.4058x over previous
import functools

import jax
import jax.numpy as jnp
from jax import lax
from jax.experimental import pallas as pl
from jax.experimental.pallas import tpu as pltpu
from jax.experimental.pallas import tpu_sc as plsc

U_N = 10000
I_N = 50000
N_UI = U_N + I_N
D = 64
BATCH = 1024
S_I = 10
S_P = 10
S_S = S_I + S_P
NFD = 6 * D
EXPD = 256
THD = 64

NC = 2
NS = 16
LANES = 16

_MESH = plsc.VectorSubcoreMesh(core_axis_name="c", subcore_axis_name="s")
_SC_PARAMS = pltpu.CompilerParams(use_tc_tiling_on_sc=False)


def _slice_sizes(n):
    per = (n // NS) & ~7
    last = n - per * (NS - 1)
    return per, last


def _zero_or_drain(src_ref, dst_ref, n, s):
    per, last = _slice_sizes(n)

    @pl.when(s < NS - 1)
    def _():
        pltpu.sync_copy(src_ref.at[pl.ds(s * per, per)],
                        dst_ref.at[pl.ds(s * per, per)])

    @pl.when(s == NS - 1)
    def _():
        pltpu.sync_copy(src_ref.at[pl.ds((NS - 1) * per, last)],
                        dst_ref.at[pl.ds((NS - 1) * per, last)])


@functools.cache
def _hist_kernel(n, rows):
    kh = 8
    groups = rows // kh

    @functools.partial(
        pl.kernel,
        out_type=jax.ShapeDtypeStruct((NC, n, 16), jnp.float32),
        mesh=_MESH,
        compiler_params=_SC_PARAMS,
        scratch_types=[
            pltpu.VMEM((kh, 128), jnp.int32),
            pltpu.VMEM((128, 16), jnp.float32),
            pltpu.VMEM_SHARED((n + 8, 16), jnp.float32),
        ],
    )
    def kern(idx_hbm, zeros_hbm, ones_hbm, out_hbm, iv, ones_v, acc):
        c = lax.axis_index("c")
        s = lax.axis_index("s")
        wid = c * NS + s
        pltpu.sync_copy(ones_hbm, ones_v)
        _zero_or_drain(zeros_hbm, acc, n, s)
        plsc.subcore_barrier()

        @pl.loop(wid, groups, step=NC * NS)
        def _(g):
            pltpu.sync_copy(idx_hbm.at[pl.ds(g * kh, kh)], iv)
            for j in range(kh):
                pltpu.sync_copy(ones_v, acc.at[iv.at[j]], add=True)

        plsc.subcore_barrier()
        _zero_or_drain(acc, out_hbm.at[c], n, s)

    return kern


@functools.cache
def _prop_kernel(n, rows):
    kh = 8
    groups = rows // kh

    @functools.partial(
        pl.kernel,
        out_type=jax.ShapeDtypeStruct((4, n, 16), jnp.float32),
        mesh=_MESH,
        compiler_params=_SC_PARAMS,
        scratch_types=[
            pltpu.VMEM((kh, 128), jnp.int32),
            pltpu.VMEM((kh, 128), jnp.int32),
            pltpu.VMEM((128, 16), jnp.float32),
            pltpu.VMEM_SHARED((n + 8, 16), jnp.float32),
        ],
    )
    def kern(hp_hbm, src_hbm, dst_hbm, zeros_hbm, out_hbm, sv, dv, rv, acc):
        c = lax.axis_index("c")
        s = lax.axis_index("s")
        for cc in range(2):
            chunk = c * 2 + cc
            _zero_or_drain(zeros_hbm, acc, n, s)
            plsc.subcore_barrier()

            @pl.loop(s, groups, step=NS)
            def _(g):
                pltpu.sync_copy(src_hbm.at[pl.ds(g * kh, kh)], sv)
                pltpu.sync_copy(dst_hbm.at[pl.ds(g * kh, kh)], dv)
                for j in range(kh):
                    pltpu.sync_copy(hp_hbm.at[chunk].at[sv.at[j]], rv)
                    pltpu.sync_copy(rv, acc.at[dv.at[j]], add=True)

            plsc.subcore_barrier()
            _zero_or_drain(acc, out_hbm.at[chunk], n, s)
            plsc.subcore_barrier()

    return kern


@functools.cache
def _sample_gather_kernel():
    jobs = (
        (0, 0, 8, (BATCH, D)),
        (2, 0, 8, (BATCH, D)),
        (0, 1, 80, (BATCH * S_I, D)),
        (1, 1, 80, (BATCH * S_I, D)),
        (1, 2, 80, (BATCH * S_P, D)),
        (2, 2, 80, (BATCH * S_P, D)),
    )

    @functools.partial(
        pl.kernel,
        out_type=[jax.ShapeDtypeStruct(j[3], jnp.float32) for j in jobs],
        mesh=_MESH,
        compiler_params=_SC_PARAMS,
        scratch_types=[
            pltpu.VMEM((80, 128), jnp.int32),
            pltpu.VMEM((128, D), jnp.float32),
        ],
    )
    def kern(t_ii, t_pi, t_ip, tu3, is3, us3, o0, o1, o2, o3, o4, o5, iv, rv):
        c = lax.axis_index("c")
        s = lax.axis_index("s")
        wid = c * NS + s
        tables = (t_ii, t_pi, t_ip)
        idxs = (tu3, is3, us3)
        outs = (o0, o1, o2, o3, o4, o5)
        for jn, (tab_i, idx_i, rows, _) in enumerate(jobs):
            tab = tables[tab_i]
            idx3 = idxs[idx_i]
            out = outs[jn]
            pltpu.sync_copy(idx3, iv.at[pl.ds(0, rows)])

            @pl.loop(wid, rows, step=NC * NS)
            def _(r):
                pltpu.sync_copy(tab.at[iv.at[r]], rv)
                pltpu.sync_copy(rv, out.at[pl.ds(r * 128, 128)])

    return kern


@functools.cache
def _scale_call(n8, final):
    def body(a_ref, b_ref, ab_ref, o_ref, out_ref, hp_ref):
        a = a_ref[...]
        res = o_ref[...] + a * b_ref[...][None]
        if final:
            res = res * 0.25
        out_ref[...] = res
        hp_ref[...] = a * ab_ref[...][None]

    return pl.pallas_call(
        body,
        grid=(4,),
        in_specs=[
            pl.BlockSpec((1, n8, 128), lambda c: (c, 0, 0)),
            pl.BlockSpec((n8, 128), lambda c: (0, 0)),
            pl.BlockSpec((n8, 128), lambda c: (0, 0)),
            pl.BlockSpec((1, n8, 128), lambda c: (c, 0, 0)),
        ],
        out_specs=[
            pl.BlockSpec((1, n8, 128), lambda c: (c, 0, 0)),
            pl.BlockSpec((1, n8, 128), lambda c: (c, 0, 0)),
        ],
        out_shape=[jax.ShapeDtypeStruct((4, n8, 128), jnp.float32)] * 2,
    )


@functools.cache
def _allp_call():
    k = 10
    br = U_N // k

    def body(pu_ref, ip_ref, o1_ref, o2_ref):
        i = pl.program_id(0)
        s1 = jnp.sum(pu_ref[...], axis=0, keepdims=True)
        s2 = jnp.sum(ip_ref[...], axis=0, keepdims=True)

        @pl.when(i == 0)
        def _():
            o1_ref[...] = s1
            o2_ref[...] = s2

        @pl.when(i > 0)
        def _():
            o1_ref[...] += s1
            o2_ref[...] += s2

        @pl.when(i == k - 1)
        def _():
            o1_ref[...] *= (1.0 / U_N)
            o2_ref[...] *= (1.0 / U_N)

    return pl.pallas_call(
        body,
        grid=(k,),
        in_specs=[
            pl.BlockSpec((br, D), lambda i: (i, 0)),
            pl.BlockSpec((br, D), lambda i: (i, 0)),
        ],
        out_specs=[
            pl.BlockSpec((1, D), lambda i: (0, 0)),
            pl.BlockSpec((1, D), lambda i: (0, 0)),
        ],
        out_shape=[jax.ShapeDtypeStruct((1, D), jnp.float32)] * 2,
    )


def _logsig(z):
    return -(jnp.log(1.0 + jnp.exp(-jnp.abs(z))) + jnp.maximum(-z, 0.0))


@functools.cache
def _mtl_call():
    bu = 64
    grid = (BATCH // bu,)

    def f32dot(a, b):
        return jax.lax.dot_general(a, b, (((a.ndim - 1,), (0,)), ((), ())),
                                   preferred_element_type=jnp.float32)

    def body(u1, u2, i1, i2, p1, p2, ap_ref,
             wse, bse, wte1, bte1, wte2, bte2,
             g1w, g1b, g2w, g2b,
             t1w1, t1b1, t1w2, t1b2, t2w1, t2b1, t2w2, t2b2,
             loss_ref, t1s_ref, t2s_ref):
        uc = jnp.concatenate([u1[...], u2[...]], axis=1)
        ic = jnp.concatenate([i1[...], i2[...]], axis=1)
        pc = jnp.concatenate([p1[...], p2[...]], axis=1)
        ap = ap_ref[...]

        def gate(gw_ref, gb_ref):
            gw = gw_ref[...]
            gb = gb_ref[...]
            ga, gbk, gc = gw[0:128], gw[128:256], gw[256:384]
            gu = f32dot(uc, ga)
            gi = f32dot(ic, gbk).reshape(bu, S_I, 6)
            gp = f32dot(pc, gc).reshape(bu, S_P, 6)
            gap = f32dot(ap, gc)
            li = gu[:, None, :] + gi + gap[None] + gb
            lp = gu[:, None, :] + gi[:, 0:1] + gp + gb
            def sm(x):
                m = jnp.max(x, axis=-1, keepdims=True)
                ex = jnp.exp(x - m)
                return ex / jnp.sum(ex, axis=-1, keepdims=True)
            return sm(li), sm(lp)

        g1i, g1p = gate(g1w, g1b)
        g2i, g2p = gate(g2w, g2b)

        h1i = jnp.zeros((bu, S_I, EXPD), jnp.float32)
        h1p = jnp.zeros((bu, S_P, EXPD), jnp.float32)
        h2i = jnp.zeros((bu, S_I, EXPD), jnp.float32)
        h2p = jnp.zeros((bu, S_P, EXPD), jnp.float32)

        for e in range(10):
            if e < 2:
                w = wse[e]
                b = bse[e]
                col = e
                br1, br2 = True, True
            elif e < 6:
                w = wte1[e - 2]
                b = bte1[e - 2]
                col = e
                br1, br2 = True, False
            else:
                w = wte2[e - 6]
                b = bte2[e - 6]
                col = e - 4
                br1, br2 = False, True
            zu = f32dot(uc, w[0:128])
            zi = f32dot(ic, w[128:256]).reshape(bu, S_I, EXPD)
            zp = f32dot(pc, w[256:384]).reshape(bu, S_P, EXPD)
            za = f32dot(ap, w[256:384])
            ei = jnp.maximum(zu[:, None, :] + zi + za[None] + b, 0.0)
            ep = jnp.maximum(zu[:, None, :] + zi[:, 0:1] + zp + b, 0.0)
            if br1:
                h1i = h1i + g1i[..., col:col + 1] * ei
                h1p = h1p + g1p[..., col:col + 1] * ep
            if br2:
                h2i = h2i + g2i[..., col:col + 1] * ei
                h2p = h2p + g2p[..., col:col + 1] * ep

        def tower(h, w1, b1, w2, b2):
            hh = jnp.maximum(f32dot(h, w1[...]) + b1[...], 0.0)
            return f32dot(hh, w2[...]) + b2[...]

        o1i = tower(h1i.reshape(bu * S_I, EXPD), t1w1, t1b1, t1w2,
                    t1b2).reshape(bu, S_I)
        o1p = tower(h1p.reshape(bu * S_P, EXPD), t1w1, t1b1, t1w2,
                    t1b2).reshape(bu, S_P)
        o2i = tower(h2i.reshape(bu * S_I, EXPD), t2w1, t2b1, t2w2,
                    t2b2).reshape(bu, S_I)
        o2p = tower(h2p.reshape(bu * S_P, EXPD), t2w1, t2b1, t2w2,
                    t2b2).reshape(bu, S_P)
        o1 = jnp.concatenate([o1i, o1p], axis=1)
        o2 = jnp.concatenate([o2i, o2p], axis=1)

        t1s = o1i
        t2s = o2p
        bpr1 = jnp.mean(-_logsig(t1s[:, 0:1] - t1s[:, 1:5]), axis=-1)
        bpr2 = jnp.mean(-_logsig(t2s[:, 0:1] - t2s[:, 1:5]), axis=-1)
        bprloss = 0.3 * bpr1 + bpr2

        m = jnp.max(o1, axis=-1, keepdims=True)
        lse = jnp.log(jnp.sum(jnp.exp(o1 - m), axis=-1)) + m[:, 0]
        lane = lax.broadcasted_iota(jnp.int32, (1, S_S), 1)
        tl = jnp.where((lane == 0) | (lane >= S_I), 1.0, 0.0)
        tsm = jnp.exp(tl) / jnp.sum(jnp.exp(tl))
        l1 = lse - jnp.sum(tsm * o1, axis=-1)

        b2v = jnp.mean(-_logsig(o2i[:, 0:1] - o2i[:, 1:S_I]), axis=-1)

        loss_ref[...] = (bprloss + 0.3 * l1 + b2v)[:, None]
        t1s_ref[...] = t1s
        t2s_ref[...] = t2s

    full = lambda shape: pl.BlockSpec(shape, lambda i: tuple(0 for _ in shape))
    return pl.pallas_call(
        body,
        grid=grid,
        in_specs=[
            pl.BlockSpec((bu, D), lambda i: (i, 0)),
            pl.BlockSpec((bu, D), lambda i: (i, 0)),
            pl.BlockSpec((bu * S_I, D), lambda i: (i, 0)),
            pl.BlockSpec((bu * S_I, D), lambda i: (i, 0)),
            pl.BlockSpec((bu * S_P, D), lambda i: (i, 0)),
            pl.BlockSpec((bu * S_P, D), lambda i: (i, 0)),
            full((1, 128)),
            full((2, NFD, EXPD)), full((2, EXPD)),
            full((4, NFD, EXPD)), full((4, EXPD)),
            full((4, NFD, EXPD)), full((4, EXPD)),
            full((NFD, 6)), full((6,)),
            full((NFD, 6)), full((6,)),
            full((EXPD, THD)), full((THD,)), full((THD, 1)), full((1,)),
            full((EXPD, THD)), full((THD,)), full((THD, 1)), full((1,)),
        ],
        out_specs=[
            pl.BlockSpec((bu, 1), lambda i: (i, 0)),
            pl.BlockSpec((bu, S_I), lambda i: (i, 0)),
            pl.BlockSpec((bu, S_P), lambda i: (i, 0)),
        ],
        out_shape=[
            jax.ShapeDtypeStruct((BATCH, 1), jnp.float32),
            jax.ShapeDtypeStruct((BATCH, S_I), jnp.float32),
            jax.ShapeDtypeStruct((BATCH, S_P), jnp.float32),
        ],
    )


def _pad_idx(idx, pad_val, rows):
    e = idx.shape[0]
    pad = rows * 128 - e
    idx = idx.astype(jnp.int32)
    if pad:
        idx = jnp.concatenate([idx, jnp.full((pad,), pad_val, jnp.int32)])
    return idx.reshape(rows, 128)


def _lightgcn(x, src, dst, n):
    e = src.shape[0]
    f32 = jnp.float32
    rows = -(-e // 1024) * 8
    src3 = _pad_idx(src, 0, rows)
    dst3 = _pad_idx(dst, n, rows)
    zeros_n16 = jnp.zeros((n, 16), f32)
    ones_128 = jnp.ones((128, 16), f32)

    hist = _hist_kernel(n, rows)
    do = hist(_pad_idx(src, n, rows), zeros_n16, ones_128)
    di = hist(dst3, zeros_n16, ones_128)
    deg_o = do[0, :, 0] + do[1, :, 0]
    deg_i = di[0, :, 0] + di[1, :, 0]
    a = 1.0 / jnp.sqrt(jnp.maximum(deg_o, 1.0))
    b = 1.0 / jnp.sqrt(jnp.maximum(deg_i, 1.0))

    n8 = n * 16 // 128
    a_b = jnp.repeat(a[:, None], 16, axis=1).reshape(n8, 128)
    b_b = jnp.repeat(b[:, None], 16, axis=1).reshape(n8, 128)
    ab_b = a_b * b_b
    ones_v = jnp.ones((n8, 128), f32)
    zeros_v4 = jnp.zeros((4, n8, 128), f32)

    x4 = x.astype(f32).reshape(n, 4, 16).transpose(1, 0, 2)
    x4v = x4.reshape(4, n8, 128)

    out4, hp = _scale_call(n8, False)(x4v, ones_v, a_b, zeros_v4)
    prop = _prop_kernel(n, rows)
    for r in range(3):
        acc4 = prop(hp.reshape(4, n, 16), src3, dst3, zeros_n16)
        out4, hp = _scale_call(n8, r == 2)(
            acc4.reshape(4, n8, 128), b_b, ab_b, out4)
    return out4.reshape(4, n, 16).transpose(1, 0, 2).reshape(n, D)


def kernel(target_user, item_sample, user_sample, ui_src, ui_dst, ui_w,
           pi_src, pi_dst, pi_w, up_src, up_dst, up_w,
           embed_W, embed_pi_W, embed_u_W, W_se, b_se, W_te1, b_te1,
           W_te2, b_te2, gate1_W, gate1_b, gate2_W, gate2_b,
           t1_W1, t1_b1, t1_W2, t1_b2, t2_W1, t2_b1, t2_W2, t2_b2):
    i32 = jnp.int32
    f32 = jnp.float32

    init_item = _lightgcn(embed_W, ui_src, ui_dst, N_UI)
    part_item = _lightgcn(embed_pi_W, pi_src, pi_dst, N_UI)
    init_part = _lightgcn(embed_u_W, up_src, up_dst, U_N)

    tu3 = target_user.astype(i32).reshape(8, 128)
    is3 = (item_sample.reshape(-1).astype(i32) + U_N).reshape(80, 128)
    us3 = user_sample.reshape(-1).astype(i32).reshape(80, 128)

    u1, u2, i1, i2, p1, p2 = _sample_gather_kernel()(
        init_item, part_item, init_part, tu3, is3, us3)

    s_pu, s_ip = _allp_call()(part_item[:U_N], init_part)
    allp128 = jnp.concatenate([s_pu, s_ip], axis=1)

    loss, t1s, t2s = _mtl_call()(
        u1, u2, i1, i2, p1, p2, allp128,
        W_se.astype(f32), b_se.astype(f32),
        W_te1.astype(f32), b_te1.astype(f32),
        W_te2.astype(f32), b_te2.astype(f32),
        gate1_W.astype(f32), gate1_b.astype(f32),
        gate2_W.astype(f32), gate2_b.astype(f32),
        t1_W1.astype(f32), t1_b1.astype(f32), t1_W2.astype(f32),
        t1_b2.astype(f32), t2_W1.astype(f32), t2_b1.astype(f32),
        t2_W2.astype(f32), t2_b2.astype(f32))
    return loss.reshape(BATCH), t1s, t2s
